# baseline (speedup 1.0000x reference)
import functools

import jax
import jax.numpy as jnp
from jax import lax
from jax.experimental import pallas as pl
from jax.experimental.pallas import tpu as pltpu
from jax.experimental.pallas import tpu_sc as plsc

_NEG_INF = float("-inf")
_GW = 32


def _scan_body(q_ref, m_ref, sb_ref, gid_ref, qn_ref, run_ref, *, Mb, K):
    j = pl.program_id(0)
    B = q_ref.shape[0]
    Gb = Mb // _GW

    @pl.when(j == 0)
    def _init():
        q = q_ref[...]
        qn_ref[...] = q / jnp.maximum(
            jnp.sqrt(jnp.sum(q * q, axis=1, keepdims=True)), 1e-12)
        run_ref[...] = jnp.full((B, K), jnp.int32(-2**31), jnp.int32)

    m = m_ref[...]
    mn = m / jnp.maximum(
        jnp.sqrt(jnp.sum(m * m, axis=1, keepdims=True)), 1e-12)
    s = lax.dot_general(qn_ref[...], mn, (((1,), (1,)), ((), ())),
                        preferred_element_type=jnp.float32)
    s = jnp.where(s > 0.999, _NEG_INF, s)
    sb_ref[...] = s.astype(jnp.bfloat16)[None]
    gmax = jnp.max(s.reshape(B, Gb, _GW), axis=2)
    bits = lax.bitcast_convert_type(gmax, jnp.int32)
    mono = bits ^ (lax.shift_right_arithmetic(bits, 31)
                   & jnp.int32(0x7FFFFFFF))
    gcol = j * Gb + lax.broadcasted_iota(jnp.int32, (B, Gb), 1)
    gkey = (mono & jnp.int32(~0x1FFF)) | (jnp.int32(8191) - gcol)

    run = run_ref[...]
    kk = lax.broadcasted_iota(jnp.int32, (B, K), 1)
    for _ in range(K):
        kmax = jnp.max(gkey, axis=1, keepdims=True)
        gkey = jnp.where(gkey == kmax, jnp.int32(-2**31), gkey)
        pos = jnp.sum((run >= kmax).astype(jnp.int32), axis=1, keepdims=True)
        sh = jnp.concatenate([run[:, :1], run[:, :K - 1]], axis=1)
        run = jnp.where(kk < pos, run, jnp.where(kk == pos, kmax, sh))
    run_ref[...] = run
    gid_ref[...] = jnp.int32(8191) - (run & jnp.int32(0x1FFF))


def _sims_scan(query, memory_bank, Mb, K):
    B, D = query.shape
    M = memory_bank.shape[0]
    nblocks = M // Mb
    body = functools.partial(_scan_body, Mb=Mb, K=K)
    return pl.pallas_call(
        body,
        grid=(nblocks,),
        in_specs=[
            pl.BlockSpec((B, D), lambda j: (0, 0)),
            pl.BlockSpec((Mb, D), lambda j: (j, 0)),
        ],
        out_specs=[
            pl.BlockSpec((1, B, Mb), lambda j: (j, 0, 0)),
            pl.BlockSpec((B, K), lambda j: (0, 0)),
        ],
        out_shape=[
            jax.ShapeDtypeStruct((nblocks, B, Mb), jnp.bfloat16),
            jax.ShapeDtypeStruct((B, K), jnp.int32),
        ],
        scratch_shapes=[pltpu.VMEM((B, D), jnp.float32),
                        pltpu.VMEM((B, K), jnp.int32)],
        compiler_params=pltpu.CompilerParams(
            dimension_semantics=("arbitrary",)),
    )(query, memory_bank)


def _sc_gather(table, idx_flat):
    Bf = idx_flat.shape[0]
    D = table.shape[1]
    info = plsc.get_sparse_core_info()
    NC, NS = info.num_cores, info.num_subcores
    NW = NC * NS
    b_per_w = Bf // NW
    mesh = plsc.VectorSubcoreMesh(core_axis_name="c", subcore_axis_name="s")

    @functools.partial(
        pl.kernel, mesh=mesh,
        out_type=jax.ShapeDtypeStruct((Bf, D), table.dtype),
        scratch_types=[
            pltpu.VMEM((b_per_w,), jnp.int32),
            pltpu.VMEM((b_per_w, D), table.dtype),
            pltpu.SemaphoreType.DMA,
        ],
    )
    def gather_k(table_hbm, idx_hbm, out_hbm, idx_v, rows_v, sem):
        wid = lax.axis_index("s") * NC + lax.axis_index("c")
        base = wid * b_per_w
        pltpu.sync_copy(idx_hbm.at[pl.ds(base, b_per_w)], idx_v)
        pltpu.async_copy(table_hbm.at[idx_v], rows_v, sem).wait()
        pltpu.sync_copy(rows_v, out_hbm.at[pl.ds(base, b_per_w)])

    return gather_k(table, idx_flat)


def _fsel_body(raw_ref, cs_ref, gid_ref, vals_ref, idx_ref, *, K):
    B = raw_ref.shape[0]
    HW = _GW // 2
    raw = raw_ref[...]
    cs = cs_ref[...]
    gid = gid_ref[...]
    picked = []
    for t in range(K):
        seg = raw[:, t * 128:(t + 1) * 128]
        acc = seg[:, 0:HW]
        for c in range(1, 8):
            acc = jnp.where(cs[:, t:t + 1] == c,
                            seg[:, c * HW:(c + 1) * HW], acc)
        picked.append(acc)
    ci = jnp.concatenate(picked, axis=1)

    def mono16(x):
        sgn = lax.shift_right_arithmetic(lax.shift_left(x, 16), 31)
        return x ^ (jnp.int32(0x8000) | (sgn & jnp.int32(0x7FFF)))

    p = lax.broadcasted_iota(jnp.int32, (B, K * HW), 1)
    cid_lo = lax.shift_right_logical(p, 4) * _GW \
        + lax.shift_left(p & jnp.int32(HW - 1), 1)
    lo = ci & jnp.int32(0xFFFF)
    hi = lax.shift_right_logical(ci, 16)
    klo = lax.shift_left(mono16(lo), 8) | (jnp.int32(255) - cid_lo)
    khi = lax.shift_left(mono16(hi), 8) | (jnp.int32(254) - cid_lo)
    key = jnp.concatenate([klo, khi], axis=1)

    vs, ix = [], []
    for _ in range(K):
        kmax = jnp.max(key, axis=1, keepdims=True)
        key = jnp.where(key == kmax, jnp.int32(-1), key)
        m16 = lax.shift_right_logical(kmax, 8)
        sgn2 = lax.shift_right_arithmetic(lax.shift_left(m16, 16), 31)
        b16 = m16 ^ (jnp.int32(0x8000) | (jnp.bitwise_not(sgn2)
                                          & jnp.int32(0x7FFF)))
        v = lax.bitcast_convert_type(lax.shift_left(b16, 16), jnp.float32)
        v = jnp.where(v >= 0.0, v, _NEG_INF)
        cid = jnp.int32(255) - (kmax & jnp.int32(0xFF))
        t = lax.shift_right_logical(cid, 5)
        l = cid & jnp.int32(_GW - 1)
        gsel = jnp.zeros_like(t)
        for tt in range(K):
            gsel = jnp.where(t == tt, gid[:, tt:tt + 1], gsel)
        vs.append(v)
        ix.append(gsel * _GW + l)
    vals_ref[...] = jnp.concatenate(vs, axis=1)
    idx_ref[...] = jnp.concatenate(ix, axis=1)


def _final_select(raw, cs, gid, K):
    B = raw.shape[0]
    body = functools.partial(_fsel_body, K=K)
    return pl.pallas_call(
        body,
        out_shape=[jax.ShapeDtypeStruct((B, K), jnp.float32),
                   jax.ShapeDtypeStruct((B, K), jnp.int32)],
    )(raw, cs, gid)


def _attn_body(q_ref, ret_ref, ts_ref, wq_ref, bq_ref, wk_ref, bk_ref,
               wv_ref, bv_ref, wo_ref, bo_ref, wg1_ref, wg2_ref, bg_ref,
               gamma_ref, beta_ref, out_ref, *, K):
    B, D = q_ref.shape
    q = q_ref[...]
    qn = q / jnp.maximum(
        jnp.sqrt(jnp.sum(q * q, axis=1, keepdims=True)), 1e-12)
    Q = jnp.dot(q, wq_ref[...], preferred_element_type=jnp.float32) \
        + bq_ref[...]
    ts = ts_ref[...]
    mask = ts > _NEG_INF
    scale = D ** -0.5

    rnorms = []
    scores = []
    sims = []
    for k in range(K):
        Rk = ret_ref[k * B:(k + 1) * B, :]
        rn = jnp.maximum(
            jnp.sqrt(jnp.sum(Rk * Rk, axis=1, keepdims=True)), 1e-12)
        rnorms.append(rn)
        Rkn = Rk / rn
        sims.append(jnp.sum(qn * Rkn, axis=1, keepdims=True))
        Kp = jnp.dot(Rkn, wk_ref[...],
                     preferred_element_type=jnp.float32) + bk_ref[...]
        scores.append(jnp.sum(Q * Kp, axis=1, keepdims=True) * scale)
    sc = jnp.concatenate(scores, axis=1)
    sc = jnp.where(mask, sc, _NEG_INF)
    mx = jnp.max(sc, axis=1, keepdims=True)
    e = jnp.where(mask, jnp.exp(sc - mx), 0.0)
    w = e / jnp.maximum(jnp.sum(e, axis=1, keepdims=True), 1e-30)
    w = jnp.where(mask, w, 0.0)

    mem = jnp.zeros((B, D), jnp.float32)
    for k in range(K):
        Rk = ret_ref[k * B:(k + 1) * B, :]
        V = jnp.dot(Rk / rnorms[k], wv_ref[...],
                    preferred_element_type=jnp.float32) + bv_ref[...]
        mem = mem + w[:, k:k + 1] * V
    mem = jnp.dot(mem, wo_ref[...], preferred_element_type=jnp.float32) \
        + bo_ref[...]

    sim = jnp.concatenate(sims, axis=1)
    max_sim = jnp.max(jnp.where(mask, sim, _NEG_INF), axis=1, keepdims=True)
    g_lin = (jnp.sum(q * wg1_ref[...], axis=1, keepdims=True)
             + jnp.sum(mem * wg2_ref[...], axis=1, keepdims=True)
             + bg_ref[...])
    gate = jax.nn.sigmoid(g_lin) * jax.nn.sigmoid(max_sim)
    out = q + gate * mem
    mu = jnp.mean(out, axis=1, keepdims=True)
    d = out - mu
    var = jnp.mean(d * d, axis=1, keepdims=True)
    out_ref[...] = d / jnp.sqrt(var + 1e-5) * gamma_ref[...] + beta_ref[...]


def _attention(query, retrieved, top_sims, Wq, bq, Wk, bk, Wv, bv, Wo, bo,
               wg1, wg2, bg, gamma, beta, K):
    B, D = query.shape
    body = functools.partial(_attn_body, K=K)
    return pl.pallas_call(
        body,
        out_shape=jax.ShapeDtypeStruct((B, D), jnp.float32),
    )(query, retrieved, top_sims, Wq, bq, Wk, bk, Wv, bv, Wo, bo,
      wg1, wg2, bg, gamma, beta)


def kernel(query, memory_bank, Wq, bq, Wk, bk, Wv, bv, Wo, bo, Wg, bg,
           gamma, beta):
    B, D = query.shape
    M = memory_bank.shape[0]
    K = 5
    G = M // _GW

    Mb = 800
    Gb = Mb // _GW
    nb = M // Mb
    sb, gid = _sims_scan(query, memory_bank, Mb, K)

    table = lax.bitcast_convert_type(
        sb.reshape(nb * B * Mb // 256, 128, 2), jnp.int32)
    brow = jnp.arange(B, dtype=jnp.int32)[:, None]
    i32idx = ((gid // Gb) * B + brow) * (Mb // 2) \
        + (gid % Gb) * (_GW // 2)
    rows = lax.shift_right_logical(i32idx, 7).reshape(-1)
    cs = lax.shift_right_logical(i32idx, 4) & 7
    raw = _sc_gather(table, rows)
    top_sims, top_idx = _final_select(raw.reshape(B, K * 128), cs, gid, K)

    idx_flat = top_idx.T.reshape(-1)
    retrieved = _sc_gather(memory_bank, idx_flat)
    wg1 = Wg[:D, 0].reshape(1, D)
    wg2 = Wg[D:, 0].reshape(1, D)
    return _attention(
        query, retrieved, top_sims, Wq, bq.reshape(1, D), Wk,
        bk.reshape(1, D), Wv, bv.reshape(1, D), Wo, bo.reshape(1, D),
        wg1, wg2, bg.reshape(1, 1), gamma.reshape(1, D), beta.reshape(1, D),
        K)


# speedup vs baseline: 36.9322x; 36.9322x over previous
import functools

import jax
import jax.numpy as jnp
from jax import lax
from jax.experimental import pallas as pl
from jax.experimental.pallas import tpu as pltpu
from jax.experimental.pallas import tpu_sc as plsc

_NEG_INF = float("-inf")


def _topk_body(q_ref, m_ref, vals_ref, idx_ref, qn_ref, kbuf_ref,
               *, M, Mb, K):
    j = pl.program_id(0)
    B = q_ref.shape[0]
    nblocks = M // Mb

    @pl.when(j == 0)
    def _init():
        q = q_ref[...]
        qn = q / jnp.maximum(
            jnp.sqrt(jnp.sum(q * q, axis=1, keepdims=True)), 1e-12)
        qn_ref[...] = qn
        vals_ref[...] = jnp.full((B, K), _NEG_INF, jnp.float32)
        idx_ref[...] = jnp.zeros((B, K), jnp.int32)

    m = m_ref[...]
    mn = m / jnp.maximum(
        jnp.sqrt(jnp.sum(m * m, axis=1, keepdims=True)), 1e-12)
    s = lax.dot_general(qn_ref[...], mn, (((1,), (1,)), ((), ())),
                        preferred_element_type=jnp.float32)
    s = jnp.where(s > 0.999, _NEG_INF, s)
    bits = lax.bitcast_convert_type(s, jnp.int32)
    mono = bits ^ (lax.shift_right_arithmetic(bits, 31)
                   & jnp.int32(0x7FFFFFFF))
    col = lax.broadcasted_iota(jnp.int32, (B, Mb), 1)
    kbuf_ref[j % 2] = (mono & jnp.int32(~0x7FF)) \
        | (jnp.int32(Mb - 1) - col)

    key = kbuf_ref[(j + 1) % 2]
    key = jnp.where(j == 0, jnp.int32(-2**31), key)
    base = (j - 1) * Mb
    vals = vals_ref[...]
    idxs = idx_ref[...]
    kk = lax.broadcasted_iota(jnp.int32, (B, K), 1)
    for _ in range(K):
        kmax = jnp.max(key, axis=1, keepdims=True)
        key = jnp.where(key == kmax, jnp.int32(-2**31), key)
        vm = kmax & jnp.int32(~0x7FF)
        vb = vm ^ (lax.shift_right_arithmetic(vm, 31) & jnp.int32(0x7FFFFFFF))
        bmf = lax.bitcast_convert_type(vb, jnp.float32)
        bmf = jnp.where(bmf >= 0.0, bmf, _NEG_INF)
        bif = (jnp.int32(Mb - 1) - (kmax & jnp.int32(0x7FF))) + base
        pos = jnp.sum((vals >= bmf).astype(jnp.int32), axis=1, keepdims=True)
        sh_vals = jnp.concatenate([vals[:, :1], vals[:, :K - 1]], axis=1)
        sh_idxs = jnp.concatenate([idxs[:, :1], idxs[:, :K - 1]], axis=1)
        vals = jnp.where(kk < pos, vals, jnp.where(kk == pos, bmf, sh_vals))
        idxs = jnp.where(kk < pos, idxs, jnp.where(kk == pos, bif, sh_idxs))
    vals_ref[...] = vals
    idx_ref[...] = idxs


def _topk_scan(query, memory_bank, K, Mb=1024):
    B, D = query.shape
    M = memory_bank.shape[0]
    nblocks = M // Mb
    assert M % Mb == 0 and Mb <= 2048
    body = functools.partial(_topk_body, M=M, Mb=Mb, K=K)
    return pl.pallas_call(
        body,
        grid=(nblocks + 1,),
        in_specs=[
            pl.BlockSpec((B, D), lambda j: (0, 0)),
            pl.BlockSpec((Mb, D), lambda j: (jnp.minimum(j, nblocks - 1), 0)),
        ],
        out_specs=[
            pl.BlockSpec((B, K), lambda j: (0, 0)),
            pl.BlockSpec((B, K), lambda j: (0, 0)),
        ],
        out_shape=[
            jax.ShapeDtypeStruct((B, K), jnp.float32),
            jax.ShapeDtypeStruct((B, K), jnp.int32),
        ],
        scratch_shapes=[pltpu.VMEM((B, D), jnp.float32),
                        pltpu.VMEM((2, B, Mb), jnp.int32)],
        compiler_params=pltpu.CompilerParams(
            dimension_semantics=("arbitrary",)),
    )(query, memory_bank)


def _sc_gather(memory_bank, idx_flat):
    Bf = idx_flat.shape[0]
    D = memory_bank.shape[1]
    info = plsc.get_sparse_core_info()
    NC, NS = info.num_cores, info.num_subcores
    NW = NC * NS
    b_per_w = Bf // NW
    mesh = plsc.VectorSubcoreMesh(core_axis_name="c", subcore_axis_name="s")

    @functools.partial(
        pl.kernel, mesh=mesh,
        out_type=jax.ShapeDtypeStruct((Bf, D), jnp.float32),
        scratch_types=[
            pltpu.VMEM((b_per_w,), jnp.int32),
            pltpu.VMEM((b_per_w, D), jnp.float32),
            pltpu.SemaphoreType.DMA,
        ],
    )
    def gather_k(table_hbm, idx_hbm, out_hbm, idx_v, rows_v, sem):
        wid = lax.axis_index("s") * NC + lax.axis_index("c")
        base = wid * b_per_w
        pltpu.sync_copy(idx_hbm.at[pl.ds(base, b_per_w)], idx_v)
        pltpu.async_copy(table_hbm.at[idx_v], rows_v, sem).wait()
        pltpu.sync_copy(rows_v, out_hbm.at[pl.ds(base, b_per_w)])

    return gather_k(memory_bank, idx_flat)


def _attn_body(q_ref, ret_ref, ts_ref, wq_ref, bq_ref, wk_ref, bk_ref,
               wv_ref, bv_ref, wo_ref, bo_ref, wg1_ref, wg2_ref, bg_ref,
               gamma_ref, beta_ref, out_ref, *, K):
    B, D = q_ref.shape
    q = q_ref[...]
    qn = q / jnp.maximum(
        jnp.sqrt(jnp.sum(q * q, axis=1, keepdims=True)), 1e-12)
    Q = jnp.dot(q, wq_ref[...], preferred_element_type=jnp.float32) \
        + bq_ref[...]
    ts = ts_ref[...]
    mask = ts > _NEG_INF
    scale = D ** -0.5

    rnorms = []
    scores = []
    sims = []
    for k in range(K):
        Rk = ret_ref[k * B:(k + 1) * B, :]
        rn = jnp.maximum(
            jnp.sqrt(jnp.sum(Rk * Rk, axis=1, keepdims=True)), 1e-12)
        rnorms.append(rn)
        Rkn = Rk / rn
        sims.append(jnp.sum(qn * Rkn, axis=1, keepdims=True))
        Kp = jnp.dot(Rkn, wk_ref[...],
                     preferred_element_type=jnp.float32) + bk_ref[...]
        scores.append(jnp.sum(Q * Kp, axis=1, keepdims=True) * scale)
    sc = jnp.concatenate(scores, axis=1)
    sc = jnp.where(mask, sc, _NEG_INF)
    mx = jnp.max(sc, axis=1, keepdims=True)
    e = jnp.where(mask, jnp.exp(sc - mx), 0.0)
    w = e / jnp.maximum(jnp.sum(e, axis=1, keepdims=True), 1e-30)
    w = jnp.where(mask, w, 0.0)

    mem = jnp.zeros((B, D), jnp.float32)
    for k in range(K):
        Rk = ret_ref[k * B:(k + 1) * B, :]
        V = jnp.dot(Rk / rnorms[k], wv_ref[...],
                    preferred_element_type=jnp.float32) + bv_ref[...]
        mem = mem + w[:, k:k + 1] * V
    mem = jnp.dot(mem, wo_ref[...], preferred_element_type=jnp.float32) \
        + bo_ref[...]

    sim = jnp.concatenate(sims, axis=1)
    max_sim = jnp.max(jnp.where(mask, sim, _NEG_INF), axis=1, keepdims=True)
    g_lin = (jnp.sum(q * wg1_ref[...], axis=1, keepdims=True)
             + jnp.sum(mem * wg2_ref[...], axis=1, keepdims=True)
             + bg_ref[...])
    gate = jax.nn.sigmoid(g_lin) * jax.nn.sigmoid(max_sim)
    out = q + gate * mem
    mu = jnp.mean(out, axis=1, keepdims=True)
    d = out - mu
    var = jnp.mean(d * d, axis=1, keepdims=True)
    out_ref[...] = d / jnp.sqrt(var + 1e-5) * gamma_ref[...] + beta_ref[...]


def _attention(query, retrieved, top_sims, Wq, bq, Wk, bk, Wv, bv, Wo, bo,
               wg1, wg2, bg, gamma, beta, K):
    B, D = query.shape
    body = functools.partial(_attn_body, K=K)
    return pl.pallas_call(
        body,
        out_shape=jax.ShapeDtypeStruct((B, D), jnp.float32),
    )(query, retrieved, top_sims, Wq, bq, Wk, bk, Wv, bv, Wo, bo,
      wg1, wg2, bg, gamma, beta)


def kernel(query, memory_bank, Wq, bq, Wk, bk, Wv, bv, Wo, bo, Wg, bg,
           gamma, beta):
    B, D = query.shape
    K = 5
    top_sims, top_idx = _topk_scan(query, memory_bank, K, Mb=2000)
    idx_flat = top_idx.T.reshape(-1)
    retrieved = _sc_gather(memory_bank, idx_flat)
    wg1 = Wg[:D, 0].reshape(1, D)
    wg2 = Wg[D:, 0].reshape(1, D)
    return _attention(
        query, retrieved, top_sims, Wq, bq.reshape(1, D), Wk,
        bk.reshape(1, D), Wv, bv.reshape(1, D), Wo, bo.reshape(1, D),
        wg1, wg2, bg.reshape(1, 1), gamma.reshape(1, D), beta.reshape(1, D),
        K)


# speedup vs baseline: 39.6219x; 1.0728x over previous
import functools

import jax
import jax.numpy as jnp
from jax import lax
from jax.experimental import pallas as pl
from jax.experimental.pallas import tpu as pltpu
from jax.experimental.pallas import tpu_sc as plsc

_NEG_INF = float("-inf")


def _topk_body(q_ref, m_ref, vals_ref, idx_ref, qn_ref, *, M, Mb, K):
    j = pl.program_id(0)
    B = q_ref.shape[0]

    @pl.when(j == 0)
    def _init():
        q = q_ref[...]
        qn = q / jnp.maximum(
            jnp.sqrt(jnp.sum(q * q, axis=1, keepdims=True)), 1e-12)
        qn_ref[...] = qn
        vals_ref[...] = jnp.full((B, K), _NEG_INF, jnp.float32)
        idx_ref[...] = jnp.zeros((B, K), jnp.int32)

    m = m_ref[...]
    mn = m / jnp.maximum(
        jnp.sqrt(jnp.sum(m * m, axis=1, keepdims=True)), 1e-12)
    s = lax.dot_general(qn_ref[...], mn, (((1,), (1,)), ((), ())),
                        preferred_element_type=jnp.float32)
    s = jnp.where(s > 0.999, _NEG_INF, s)
    bits = lax.bitcast_convert_type(s, jnp.int32)
    mono = bits ^ (lax.shift_right_arithmetic(bits, 31)
                   & jnp.int32(0x7FFFFFFF))
    col = lax.broadcasted_iota(jnp.int32, (B, Mb), 1)
    key = (mono & jnp.int32(~0x7FF)) | (jnp.int32(Mb - 1) - col)
    base = j * Mb
    vals = vals_ref[...]
    idxs = idx_ref[...]
    kk = lax.broadcasted_iota(jnp.int32, (B, K), 1)
    for _ in range(K):
        kmax = jnp.max(key, axis=1, keepdims=True)
        key = jnp.where(key == kmax, jnp.int32(-2**31), key)
        vm = kmax & jnp.int32(~0x7FF)
        vb = vm ^ (lax.shift_right_arithmetic(vm, 31) & jnp.int32(0x7FFFFFFF))
        bmf = lax.bitcast_convert_type(vb, jnp.float32)
        bmf = jnp.where(bmf >= 0.0, bmf, _NEG_INF)
        bif = (jnp.int32(Mb - 1) - (kmax & jnp.int32(0x7FF))) + base
        pos = jnp.sum((vals >= bmf).astype(jnp.int32), axis=1, keepdims=True)
        sh_vals = jnp.concatenate([vals[:, :1], vals[:, :K - 1]], axis=1)
        sh_idxs = jnp.concatenate([idxs[:, :1], idxs[:, :K - 1]], axis=1)
        vals = jnp.where(kk < pos, vals, jnp.where(kk == pos, bmf, sh_vals))
        idxs = jnp.where(kk < pos, idxs, jnp.where(kk == pos, bif, sh_idxs))
    vals_ref[...] = vals
    idx_ref[...] = idxs


def _topk_scan(query, memory_bank, K, Mb=1024):
    B, D = query.shape
    M = memory_bank.shape[0]
    nblocks = M // Mb
    assert M % Mb == 0 and Mb <= 2048
    body = functools.partial(_topk_body, M=M, Mb=Mb, K=K)
    return pl.pallas_call(
        body,
        grid=(nblocks,),
        in_specs=[
            pl.BlockSpec((B, D), lambda j: (0, 0)),
            pl.BlockSpec((Mb, D), lambda j: (j, 0)),
        ],
        out_specs=[
            pl.BlockSpec((B, K), lambda j: (0, 0)),
            pl.BlockSpec((B, K), lambda j: (0, 0)),
        ],
        out_shape=[
            jax.ShapeDtypeStruct((B, K), jnp.float32),
            jax.ShapeDtypeStruct((B, K), jnp.int32),
        ],
        scratch_shapes=[pltpu.VMEM((B, D), jnp.float32)],
        compiler_params=pltpu.CompilerParams(
            dimension_semantics=("arbitrary",)),
    )(query, memory_bank)


def _sc_gather(memory_bank, idx_flat):
    Bf = idx_flat.shape[0]
    D = memory_bank.shape[1]
    info = plsc.get_sparse_core_info()
    NC, NS = info.num_cores, info.num_subcores
    NW = NC * NS
    b_per_w = Bf // NW
    mesh = plsc.VectorSubcoreMesh(core_axis_name="c", subcore_axis_name="s")

    @functools.partial(
        pl.kernel, mesh=mesh,
        out_type=jax.ShapeDtypeStruct((Bf, D), jnp.float32),
        scratch_types=[
            pltpu.VMEM((b_per_w,), jnp.int32),
            pltpu.VMEM((b_per_w, D), jnp.float32),
            pltpu.SemaphoreType.DMA,
        ],
    )
    def gather_k(table_hbm, idx_hbm, out_hbm, idx_v, rows_v, sem):
        wid = lax.axis_index("s") * NC + lax.axis_index("c")
        base = wid * b_per_w
        pltpu.sync_copy(idx_hbm.at[pl.ds(base, b_per_w)], idx_v)
        pltpu.async_copy(table_hbm.at[idx_v], rows_v, sem).wait()
        pltpu.sync_copy(rows_v, out_hbm.at[pl.ds(base, b_per_w)])

    return gather_k(memory_bank, idx_flat)


def _attn_body(q_ref, ret_ref, ts_ref, wq_ref, bq_ref, wk_ref, bk_ref,
               wv_ref, bv_ref, wo_ref, bo_ref, wg1_ref, wg2_ref, bg_ref,
               gamma_ref, beta_ref, out_ref, *, K):
    B, D = q_ref.shape
    q = q_ref[...]
    qn = q / jnp.maximum(
        jnp.sqrt(jnp.sum(q * q, axis=1, keepdims=True)), 1e-12)
    Q = jnp.dot(q, wq_ref[...], preferred_element_type=jnp.float32) \
        + bq_ref[...]
    ts = ts_ref[...]
    mask = ts > _NEG_INF
    scale = D ** -0.5

    rnorms = []
    scores = []
    sims = []
    for k in range(K):
        Rk = ret_ref[k * B:(k + 1) * B, :]
        rn = jnp.maximum(
            jnp.sqrt(jnp.sum(Rk * Rk, axis=1, keepdims=True)), 1e-12)
        rnorms.append(rn)
        Rkn = Rk / rn
        sims.append(jnp.sum(qn * Rkn, axis=1, keepdims=True))
        Kp = jnp.dot(Rkn, wk_ref[...],
                     preferred_element_type=jnp.float32) + bk_ref[...]
        scores.append(jnp.sum(Q * Kp, axis=1, keepdims=True) * scale)
    sc = jnp.concatenate(scores, axis=1)
    sc = jnp.where(mask, sc, _NEG_INF)
    mx = jnp.max(sc, axis=1, keepdims=True)
    e = jnp.where(mask, jnp.exp(sc - mx), 0.0)
    w = e / jnp.maximum(jnp.sum(e, axis=1, keepdims=True), 1e-30)
    w = jnp.where(mask, w, 0.0)

    mem = jnp.zeros((B, D), jnp.float32)
    for k in range(K):
        Rk = ret_ref[k * B:(k + 1) * B, :]
        V = jnp.dot(Rk / rnorms[k], wv_ref[...],
                    preferred_element_type=jnp.float32) + bv_ref[...]
        mem = mem + w[:, k:k + 1] * V
    mem = jnp.dot(mem, wo_ref[...], preferred_element_type=jnp.float32) \
        + bo_ref[...]

    sim = jnp.concatenate(sims, axis=1)
    max_sim = jnp.max(jnp.where(mask, sim, _NEG_INF), axis=1, keepdims=True)
    g_lin = (jnp.sum(q * wg1_ref[...], axis=1, keepdims=True)
             + jnp.sum(mem * wg2_ref[...], axis=1, keepdims=True)
             + bg_ref[...])
    gate = jax.nn.sigmoid(g_lin) * jax.nn.sigmoid(max_sim)
    out = q + gate * mem
    mu = jnp.mean(out, axis=1, keepdims=True)
    d = out - mu
    var = jnp.mean(d * d, axis=1, keepdims=True)
    out_ref[...] = d / jnp.sqrt(var + 1e-5) * gamma_ref[...] + beta_ref[...]


def _attention(query, retrieved, top_sims, Wq, bq, Wk, bk, Wv, bv, Wo, bo,
               wg1, wg2, bg, gamma, beta, K):
    B, D = query.shape
    body = functools.partial(_attn_body, K=K)
    return pl.pallas_call(
        body,
        out_shape=jax.ShapeDtypeStruct((B, D), jnp.float32),
    )(query, retrieved, top_sims, Wq, bq, Wk, bk, Wv, bv, Wo, bo,
      wg1, wg2, bg, gamma, beta)


def kernel(query, memory_bank, Wq, bq, Wk, bk, Wv, bv, Wo, bo, Wg, bg,
           gamma, beta):
    B, D = query.shape
    K = 5
    top_sims, top_idx = _topk_scan(query, memory_bank, K, Mb=2000)
    idx_flat = top_idx.T.reshape(-1)
    retrieved = _sc_gather(memory_bank, idx_flat)
    wg1 = Wg[:D, 0].reshape(1, D)
    wg2 = Wg[D:, 0].reshape(1, D)
    return _attention(
        query, retrieved, top_sims, Wq, bq.reshape(1, D), Wk,
        bk.reshape(1, D), Wv, bv.reshape(1, D), Wo, bo.reshape(1, D),
        wg1, wg2, bg.reshape(1, 1), gamma.reshape(1, D), beta.reshape(1, D),
        K)
